# baseline (speedup 1.0000x reference)
import functools

import jax
import jax.numpy as jnp
from jax import lax
from jax.experimental import pallas as pl
from jax.experimental.pallas import tpu as pltpu
from jax.experimental.pallas import tpu_sc as plsc

V = 30522
D = 768
B = 16384
L = 32
LANES = 16
NC = 2
NS = 16
NW = NC * NS
BPW = B // NW
QC = 16
NCHUNK = BPW // QC
SEC = 3
SECW = D // SEC

_mesh = plsc.VectorSubcoreMesh(core_axis_name="c", subcore_axis_name="s")


@functools.partial(
    pl.kernel,
    out_type=jax.ShapeDtypeStruct((B, D), jnp.float32),
    mesh=_mesh,
    scratch_types=[
        pltpu.VMEM((BPW * L,), jnp.int32),
        pltpu.VMEM((BPW * L,), jnp.int32),
        pltpu.VMEM((V,), jnp.float32),
        pltpu.VMEM((4, L // 2, D), jnp.float32),
        pltpu.VMEM((QC, D), jnp.float32),
        pltpu.VMEM((L,), jnp.float32),
        pltpu.SemaphoreType.DMA,
        pltpu.SemaphoreType.DMA,
        pltpu.SemaphoreType.DMA,
        pltpu.SemaphoreType.DMA,
    ],
    compiler_params=pltpu.CompilerParams(needs_layout_passes=False),
)
def _sc_avg_emb(ids_hbm, mask_hbm, temb_hbm, tw_hbm, out_hbm,
                ids_v, mask_v, tw_v, rows_v, out_v, w_v,
                gsem0, gsem1, gsem2, gsem3):
    wid = lax.axis_index("s") * NC + lax.axis_index("c")
    base = wid * BPW

    pltpu.sync_copy(ids_hbm.at[pl.ds(base * L, BPW * L)], ids_v)
    pltpu.sync_copy(mask_hbm.at[pl.ds(base * L, BPW * L)], mask_v)
    pltpu.sync_copy(tw_hbm, tw_v)

    H = L // 2

    def issue(q, slot, sema, semb):
        pltpu.async_copy(temb_hbm.at[ids_v.at[pl.ds(q * L, H)]],
                         rows_v.at[2 * slot], sema)
        pltpu.async_copy(temb_hbm.at[ids_v.at[pl.ds(q * L + H, H)]],
                         rows_v.at[2 * slot + 1], semb)

    def wait(slot, sema, semb):
        pltpu.make_async_copy(temb_hbm.at[pl.ds(0, H)], rows_v.at[2 * slot],
                              sema).wait()
        pltpu.make_async_copy(temb_hbm.at[pl.ds(0, H)],
                              rows_v.at[2 * slot + 1], semb).wait()

    def compute(q, slot, qo):
        i0 = ids_v[pl.ds(q * L, LANES)]
        i1 = ids_v[pl.ds(q * L + LANES, LANES)]
        v0 = plsc.load_gather(tw_v, [i0])
        v1 = plsc.load_gather(tw_v, [i1])
        m = jnp.maximum(jnp.max(v0), jnp.max(v1))
        mk0 = mask_v[pl.ds(q * L, LANES)].astype(jnp.float32)
        mk1 = mask_v[pl.ds(q * L + LANES, LANES)].astype(jnp.float32)
        e0 = jnp.exp(v0 - m) * mk0
        e1 = jnp.exp(v1 - m) * mk1
        s = jnp.full((LANES,), jnp.sum(e0) + jnp.sum(e1), jnp.float32)
        inv = 1.0 / s
        w_v[pl.ds(0, LANES)] = e0 * inv
        w_v[pl.ds(LANES, LANES)] = e1 * inv

        for half in range(2):
            hb = 2 * slot + half
            for sec in range(SEC):
                col0 = sec * SECW
                nt = SECW // LANES

                def body(l, accs):
                    w = plsc.load_gather(
                        w_v, [jnp.full((LANES,), half * H + l, jnp.int32)])
                    return tuple(
                        accs[t] + rows_v[hb, l, pl.ds(col0 + t * LANES, LANES)] * w
                        for t in range(nt))

                accs = lax.fori_loop(
                    0, H, body,
                    tuple((jnp.zeros((LANES,), jnp.float32) if half == 0 else
                           out_v[qo, pl.ds(col0 + t * LANES, LANES)])
                          for t in range(nt)))
                for t in range(nt):
                    out_v[qo, pl.ds(col0 + t * LANES, LANES)] = accs[t]

    issue(0, 0, gsem0, gsem1)

    def pair_body(p, _):
        q0 = 2 * p
        issue(q0 + 1, 1, gsem2, gsem3)
        wait(0, gsem0, gsem1)
        compute(q0, 0, lax.rem(q0, QC))

        @pl.when(q0 + 2 < BPW)
        def _():
            issue(q0 + 2, 0, gsem0, gsem1)

        wait(1, gsem2, gsem3)
        compute(q0 + 1, 1, lax.rem(q0 + 1, QC))

        @pl.when(lax.rem(p, QC // 2) == QC // 2 - 1)
        def _():
            c = p // (QC // 2)
            pltpu.sync_copy(out_v, out_hbm.at[pl.ds(base + c * QC, QC)])

        return 0

    lax.fori_loop(0, BPW // 2, pair_body, 0)


def kernel(input_ids, attention_mask, tok_embs, tok_embs_weights):
    return _sc_avg_emb(input_ids.reshape(B * L), attention_mask.reshape(B * L),
                       tok_embs, tok_embs_weights)


# speedup vs baseline: 1.0595x; 1.0595x over previous
import functools

import jax
import jax.numpy as jnp
from jax import lax
from jax.experimental import pallas as pl
from jax.experimental.pallas import tpu as pltpu
from jax.experimental.pallas import tpu_sc as plsc

V = 30522
D = 768
B = 16384
L = 32
LANES = 16
NC = 2
NS = 16
NW = NC * NS
BPW = B // NW
QC = 16
NCHUNK = BPW // QC
SEC = 3
SECW = D // SEC
H = L // 2
DW = D // 2
NJ = D // 32

CHK = 32
WPR = 960
NCH = WPR // CHK
CLAMP = 30488
VTAIL = 30520

_mesh = plsc.VectorSubcoreMesh(core_axis_name="c", subcore_axis_name="s")
_PACK = plsc.PackFormat.INTERLEAVED


@functools.partial(
    pl.kernel,
    out_type=jax.ShapeDtypeStruct((V, DW), jnp.int32),
    mesh=_mesh,
    scratch_types=[
        pltpu.VMEM((2, CHK, D), jnp.float32),
        pltpu.VMEM((2, CHK, DW), jnp.int32),
        pltpu.SemaphoreType.DMA,
        pltpu.SemaphoreType.DMA,
        pltpu.SemaphoreType.DMA,
        pltpu.SemaphoreType.DMA,
    ],
    compiler_params=pltpu.CompilerParams(needs_layout_passes=False),
)
def _sc_pack_table(temb_hbm, packed_hbm, fin_v, pout_v,
                   isem0, isem1, osem0, osem1):
    wid = lax.axis_index("s") * NC + lax.axis_index("c")
    start = wid * WPR

    def cstart(k):
        return jnp.minimum(start + k * CHK, CLAMP)

    def issue_in(k, slot, sem):
        pltpu.async_copy(temb_hbm.at[pl.ds(cstart(k), CHK)], fin_v.at[slot],
                         sem)

    def wait_in(slot, sem):
        pltpu.make_async_copy(temb_hbm.at[pl.ds(0, CHK)], fin_v.at[slot],
                              sem).wait()

    def issue_out(k, slot, sem):
        pltpu.async_copy(pout_v.at[slot], packed_hbm.at[pl.ds(cstart(k), CHK)],
                         sem)

    def wait_out(slot, sem):
        pltpu.make_async_copy(pout_v.at[slot], packed_hbm.at[pl.ds(0, CHK)],
                              sem).wait()

    def convert(k, slot, isem, osem):
        wait_in(slot, isem)

        @pl.when(k >= 2)
        def _():
            wait_out(slot, osem)

        def row_body(r, _):
            for j in range(NJ):
                a = fin_v[slot, r, pl.ds(32 * j, LANES)]
                b = fin_v[slot, r, pl.ds(32 * j + LANES, LANES)]
                packed = plsc.pack(a, b, format=_PACK)
                pout_v[slot, r, pl.ds(16 * j, LANES)] = plsc.bitcast(
                    packed, jnp.int32)
            return 0

        lax.fori_loop(0, CHK, row_body, 0)
        issue_out(k, slot, osem)

    issue_in(0, 0, isem0)

    def pair_body(p, _):
        k0 = 2 * p
        issue_in(k0 + 1, 1, isem1)
        convert(k0, 0, isem0, osem0)

        @pl.when(k0 + 2 < NCH)
        def _():
            issue_in(k0 + 2, 0, isem0)

        convert(k0 + 1, 1, isem1, osem1)
        return 0

    lax.fori_loop(0, NCH // 2, pair_body, 0)
    wait_out(0, osem0)
    wait_out(1, osem1)

    @pl.when(wid == 0)
    def _():
        pltpu.sync_copy(temb_hbm.at[pl.ds(VTAIL, V - VTAIL)],
                        fin_v.at[0, pl.ds(0, V - VTAIL)])

        def row_body(r, _):
            for j in range(NJ):
                a = fin_v[0, r, pl.ds(32 * j, LANES)]
                b = fin_v[0, r, pl.ds(32 * j + LANES, LANES)]
                packed = plsc.pack(a, b, format=_PACK)
                pout_v[0, r, pl.ds(16 * j, LANES)] = plsc.bitcast(
                    packed, jnp.int32)
            return 0

        lax.fori_loop(0, V - VTAIL, row_body, 0)
        pltpu.sync_copy(pout_v.at[0, pl.ds(0, V - VTAIL)],
                        packed_hbm.at[pl.ds(VTAIL, V - VTAIL)])


@functools.partial(
    pl.kernel,
    out_type=jax.ShapeDtypeStruct((B, D), jnp.float32),
    mesh=_mesh,
    scratch_types=[
        pltpu.VMEM((BPW * L,), jnp.int32),
        pltpu.VMEM((BPW * L,), jnp.int32),
        pltpu.VMEM((V,), jnp.float32),
        pltpu.VMEM((4, H, DW), jnp.int32),
        pltpu.VMEM((QC, D), jnp.float32),
        pltpu.VMEM((L,), jnp.float32),
        pltpu.SemaphoreType.DMA,
        pltpu.SemaphoreType.DMA,
        pltpu.SemaphoreType.DMA,
        pltpu.SemaphoreType.DMA,
    ],
    compiler_params=pltpu.CompilerParams(needs_layout_passes=False),
)
def _sc_avg_emb(ids_hbm, mask_hbm, temb_hbm, tw_hbm, out_hbm,
                ids_v, mask_v, tw_v, rows_v, out_v, w_v,
                gsem0, gsem1, gsem2, gsem3):
    wid = lax.axis_index("s") * NC + lax.axis_index("c")
    base = wid * BPW

    pltpu.sync_copy(ids_hbm.at[pl.ds(base * L, BPW * L)], ids_v)
    pltpu.sync_copy(mask_hbm.at[pl.ds(base * L, BPW * L)], mask_v)
    pltpu.sync_copy(tw_hbm, tw_v)

    def issue(q, slot, sema, semb):
        pltpu.async_copy(temb_hbm.at[ids_v.at[pl.ds(q * L, H)]],
                         rows_v.at[2 * slot], sema)
        pltpu.async_copy(temb_hbm.at[ids_v.at[pl.ds(q * L + H, H)]],
                         rows_v.at[2 * slot + 1], semb)

    def wait(slot, sema, semb):
        pltpu.make_async_copy(temb_hbm.at[pl.ds(0, H)], rows_v.at[2 * slot],
                              sema).wait()
        pltpu.make_async_copy(temb_hbm.at[pl.ds(0, H)],
                              rows_v.at[2 * slot + 1], semb).wait()

    def compute(q, slot, qo):
        i0 = ids_v[pl.ds(q * L, LANES)]
        i1 = ids_v[pl.ds(q * L + LANES, LANES)]
        v0 = plsc.load_gather(tw_v, [i0])
        v1 = plsc.load_gather(tw_v, [i1])
        m = jnp.maximum(jnp.max(v0), jnp.max(v1))
        mk0 = mask_v[pl.ds(q * L, LANES)].astype(jnp.float32)
        mk1 = mask_v[pl.ds(q * L + LANES, LANES)].astype(jnp.float32)
        e0 = jnp.exp(v0 - m) * mk0
        e1 = jnp.exp(v1 - m) * mk1
        s = jnp.full((LANES,), jnp.sum(e0) + jnp.sum(e1), jnp.float32)
        inv = 1.0 / s
        w_v[pl.ds(0, LANES)] = e0 * inv
        w_v[pl.ds(LANES, LANES)] = e1 * inv

        for sec in range(SEC):
            j0 = sec * (NJ // SEC)
            nt8 = NJ // SEC

            def body(l, accs):
                w = plsc.load_gather(w_v, [jnp.full((LANES,), l, jnp.int32)])
                hb = 2 * slot + l // H
                r = lax.rem(l, H)
                news = []
                for t8 in range(nt8):
                    x = rows_v[hb, r, pl.ds((j0 + t8) * LANES, LANES)]
                    aa, bb = plsc.unpack(plsc.bitcast(x, jnp.bfloat16),
                                         format=_PACK)
                    news.append(accs[2 * t8] + aa * w)
                    news.append(accs[2 * t8 + 1] + bb * w)
                return tuple(news)

            accs = lax.fori_loop(
                0, L, body,
                tuple(jnp.zeros((LANES,), jnp.float32)
                      for _ in range(2 * nt8)))
            for t in range(2 * nt8):
                out_v[qo, pl.ds((2 * j0 + t) * LANES, LANES)] = accs[t]

    issue(0, 0, gsem0, gsem1)

    def pair_body(p, _):
        q0 = 2 * p
        issue(q0 + 1, 1, gsem2, gsem3)
        wait(0, gsem0, gsem1)
        compute(q0, 0, lax.rem(q0, QC))

        @pl.when(q0 + 2 < BPW)
        def _():
            issue(q0 + 2, 0, gsem0, gsem1)

        wait(1, gsem2, gsem3)
        compute(q0 + 1, 1, lax.rem(q0 + 1, QC))

        @pl.when(lax.rem(p, QC // 2) == QC // 2 - 1)
        def _():
            c = p // (QC // 2)
            pltpu.sync_copy(out_v, out_hbm.at[pl.ds(base + c * QC, QC)])

        return 0

    lax.fori_loop(0, BPW // 2, pair_body, 0)


def kernel(input_ids, attention_mask, tok_embs, tok_embs_weights):
    packed = _sc_pack_table(tok_embs)
    return _sc_avg_emb(input_ids.reshape(B * L), attention_mask.reshape(B * L),
                       packed, tok_embs_weights)
